# Initial kernel scaffold; baseline (speedup 1.0000x reference)
import jax
import jax.numpy as jnp
from jax.experimental import pallas as pl


def kernel(point_pos):
    raise NotImplementedError("write your pallas kernel here")


# speedup vs baseline: 11.7673x; 11.7673x over previous
import jax
import jax.numpy as jnp
from jax.experimental import pallas as pl

_GRID = 8
_DIM = 3
_A = 16
_S = _GRID ** 3
_R = 2048


def _grid_init_kernel(pts_t_ref, px_t_ref):
    mn = jnp.min(pts_t_ref[...], axis=1, keepdims=True)
    mx = jnp.max(pts_t_ref[...], axis=1, keepdims=True)
    ge = (mx - mn) / jnp.float32(_GRID) * jnp.float32(0.5)
    r = jax.lax.broadcasted_iota(jnp.int32, (_DIM, _S), 0)
    s = jax.lax.broadcasted_iota(jnp.int32, (_DIM, _S), 1)
    mesh = jnp.where(r == 0, s // (_GRID * _GRID),
                     jnp.where(r == 1, (s // _GRID) % _GRID, s % _GRID))
    mesh_ph = mesh.astype(jnp.float32) + jnp.float32(0.5)
    px_t_ref[...] = mesh_ph * ge * jnp.float32(2.0) + mn


def _knn_kernel(pts_ref, px_t_ref, idx_ref):
    x = pts_ref[:, 0:1]
    y = pts_ref[:, 1:2]
    z = pts_ref[:, 2:3]
    ax = px_t_ref[0:1, :]
    ay = px_t_ref[1:2, :]
    az = px_t_ref[2:3, :]
    pt_sq = x * x + y * y + z * z
    px_sq = ax * ax + ay * ay + az * az
    dot = x * ax + y * ay + z * az
    d2 = pt_sq + px_sq - jnp.float32(2.0) * dot
    lane = jax.lax.broadcasted_iota(jnp.int32, (_R, _S), 1)
    key = (jax.lax.bitcast_convert_type(d2, jnp.int32) & jnp.int32(-512)) | lane
    cols = []
    for _ in range(_A):
        m = jnp.min(key, axis=1, keepdims=True)
        cols.append(m & jnp.int32(_S - 1))
        key = jnp.where(key == m, jnp.int32(0x7FFFFFFF), key)
    idx_ref[...] = jnp.concatenate(cols, axis=1)


def kernel(point_pos):
    p = point_pos.shape[0]
    blocks = pl.cdiv(p, _R)
    p_pad = blocks * _R
    pts = jnp.pad(point_pos, ((0, p_pad - p), (0, 0)), mode="edge")
    pts_t = pts.T

    px_t = pl.pallas_call(
        _grid_init_kernel,
        out_shape=jax.ShapeDtypeStruct((_DIM, _S), jnp.float32),
    )(pts_t)

    idx = pl.pallas_call(
        _knn_kernel,
        grid=(blocks,),
        in_specs=[
            pl.BlockSpec((_R, _DIM), lambda i: (i, 0)),
            pl.BlockSpec((_DIM, _S), lambda i: (0, 0)),
        ],
        out_specs=pl.BlockSpec((_R, _A), lambda i: (i, 0)),
        out_shape=jax.ShapeDtypeStruct((p_pad, _A), jnp.int32),
    )(pts, px_t)

    px_pos = px_t.T
    pt_ids = jnp.repeat(jnp.arange(p, dtype=jnp.int32), _A)
    px_ids = idx[:p].reshape(-1)
    assoc = jnp.stack([pt_ids, px_ids], axis=-1)
    return px_pos, assoc
